# baseline (speedup 1.0000x reference)
import functools

import jax
import jax.numpy as jnp
from jax import lax
from jax.experimental import pallas as pl
from jax.experimental.pallas import tpu as pltpu
from jax.experimental.pallas import tpu_sc as plsc

VOCAB = 100000
D = 64
B = 16384
L = 50

NC = 2
NS = 16
LANES = 16
NW = NC * NS
BPW = B // NW
IPW = BPW * L

CB = 8
NCHUNK = BPW // CB
NSUPER = NCHUNK // 2
CI = CB * L
GSIZES = [128, 128, 128, 16]

CB2 = 16
NCHUNK2 = BPW // CB2
NSUPER2 = NCHUNK2 // 2
CI2 = CB2 * L
GSIZES2 = [128] * 6 + [32]

RS_BLK = 4000
RS_GRID = 25


def _rowsum_table(vectors):
    def body(v_ref, o_ref):
        splat_p = (lax.broadcasted_iota(jnp.int32, (8, 128), 1) // LANES
                   == lax.broadcasted_iota(jnp.int32, (8, 128), 0)
                   ).astype(jnp.float32)
        v3 = v_ref[...].reshape(RS_BLK // 8, 8, D)
        s2 = jnp.sum(v3, axis=2)
        o_ref[...] = jax.lax.dot_general(
            s2, splat_p, (((1,), (0,)), ((), ())),
            preferred_element_type=jnp.float32).reshape(1, RS_BLK // 8, 128)

    rs = pl.pallas_call(
        body,
        grid=(RS_GRID,),
        in_specs=[pl.BlockSpec((RS_BLK, D), lambda i: (i, 0))],
        out_specs=pl.BlockSpec((1, RS_BLK // 8, 128), lambda i: (i, 0, 0)),
        out_shape=jax.ShapeDtypeStruct((RS_GRID, RS_BLK // 8, 128), jnp.float32),
    )(vectors)
    return rs.reshape(RS_GRID * RS_BLK, LANES)


def _sum_body(x_hbm, vec_hbm, out_hbm,
              idx_v, rows_v, stage_v, sem0, sem1, osem0, osem1):
    c = lax.axis_index("c")
    s = lax.axis_index("s")
    wid = s * NC + c
    base_b = wid * BPW

    pltpu.sync_copy(x_hbm.at[pl.ds(wid * IPW, IPW)], idx_v)

    sems = [sem0, sem1]
    osems = [osem0, osem1]

    def fire(buf, ci):
        o = 0
        for g in GSIZES:
            pltpu.async_copy(vec_hbm.at[idx_v.at[pl.ds(ci * CI + o, g)]],
                             rows_v.at[pl.ds(buf * CI + o, g)], sems[buf])
            o += g

    def drain(buf):
        pltpu.make_async_copy(vec_hbm.at[pl.ds(0, CI)],
                              rows_v.at[pl.ds(buf * CI, CI)], sems[buf]).wait()

    zero = jnp.zeros((LANES,), jnp.float32)

    def compute(buf, ci):
        def row_body(b, carry):
            r0 = buf * CI + b * L
            accs = [zero for _ in range(D // LANES)]
            for l in range(L):
                for d in range(D // LANES):
                    accs[d] = accs[d] + rows_v[r0 + l, pl.ds(d * LANES, LANES)]
            for d in range(D // LANES):
                stage_v[buf * CB + b, pl.ds(d * LANES, LANES)] = accs[d]
            return carry

        @pl.when(ci >= 2)
        def _():
            pltpu.make_async_copy(stage_v.at[pl.ds(buf * CB, CB)],
                                  out_hbm.at[pl.ds(0, CB)], osems[buf]).wait()

        lax.fori_loop(0, CB, row_body, 0)
        pltpu.async_copy(stage_v.at[pl.ds(buf * CB, CB)],
                         out_hbm.at[pl.ds(base_b + ci * CB, CB)], osems[buf])

    fire(0, 0)

    def super_body(sc, carry):
        ci0 = sc * 2
        fire(1, ci0 + 1)
        drain(0)
        compute(0, ci0)

        @pl.when(sc + 1 < NSUPER)
        def _():
            fire(0, ci0 + 2)

        drain(1)
        compute(1, ci0 + 1)
        return carry

    lax.fori_loop(0, NSUPER, super_body, 0)

    for buf in range(2):
        pltpu.make_async_copy(stage_v.at[pl.ds(buf * CB, CB)],
                              out_hbm.at[pl.ds(0, CB)], osems[buf]).wait()


def _div_body(x_hbm, rs_hbm, sums_hbm, out_hbm,
              idx_v, rsg_v, sums_v, stage_v, sem0, sem1, osem0, osem1):
    c = lax.axis_index("c")
    s = lax.axis_index("s")
    wid = s * NC + c
    base_b = wid * BPW

    pltpu.sync_copy(x_hbm.at[pl.ds(wid * IPW, IPW)], idx_v)
    pltpu.sync_copy(sums_hbm.at[pl.ds(base_b, BPW)], sums_v)

    sems = [sem0, sem1]
    osems = [osem0, osem1]

    def fire(buf, ci):
        o = 0
        for g in GSIZES2:
            pltpu.async_copy(rs_hbm.at[idx_v.at[pl.ds(ci * CI2 + o, g)]],
                             rsg_v.at[pl.ds(buf * CI2 + o, g)], sems[buf])
            o += g

    def drain(buf):
        pltpu.make_async_copy(rs_hbm.at[pl.ds(0, CI2)],
                              rsg_v.at[pl.ds(buf * CI2, CI2)], sems[buf]).wait()

    one = jnp.ones((LANES,), jnp.float32)
    zero = jnp.zeros((LANES,), jnp.float32)

    def compute(buf, ci):
        def row_body(b, carry):
            r0 = buf * CI2 + b * L
            cnt = zero
            for l in range(L):
                rsl = rsg_v[r0 + l, pl.ds(0, LANES)]
                cnt = cnt + jnp.where(rsl != 0.0, one, zero)
            inv = 1.0 / cnt
            sb = ci * CB2 + b
            for d in range(D // LANES):
                stage_v[buf * CB2 + b, pl.ds(d * LANES, LANES)] = (
                    sums_v[sb, pl.ds(d * LANES, LANES)] * inv)
            return carry

        @pl.when(ci >= 2)
        def _():
            pltpu.make_async_copy(stage_v.at[pl.ds(buf * CB2, CB2)],
                                  out_hbm.at[pl.ds(0, CB2)], osems[buf]).wait()

        lax.fori_loop(0, CB2, row_body, 0)
        pltpu.async_copy(stage_v.at[pl.ds(buf * CB2, CB2)],
                         out_hbm.at[pl.ds(base_b + ci * CB2, CB2)], osems[buf])

    fire(0, 0)

    def super_body(sc, carry):
        ci0 = sc * 2
        fire(1, ci0 + 1)
        drain(0)
        compute(0, ci0)

        @pl.when(sc + 1 < NSUPER2)
        def _():
            fire(0, ci0 + 2)

        drain(1)
        compute(1, ci0 + 1)
        return carry

    lax.fori_loop(0, NSUPER2, super_body, 0)

    for buf in range(2):
        pltpu.make_async_copy(stage_v.at[pl.ds(buf * CB2, CB2)],
                              out_hbm.at[pl.ds(0, CB2)], osems[buf]).wait()


_MESH = plsc.VectorSubcoreMesh(core_axis_name="c", subcore_axis_name="s")
_PARAMS = pltpu.CompilerParams(use_tc_tiling_on_sc=False)

_sc_sum = functools.partial(
    pl.kernel,
    out_type=jax.ShapeDtypeStruct((B, D), jnp.float32),
    mesh=_MESH,
    compiler_params=_PARAMS,
    scratch_types=[
        pltpu.VMEM((IPW,), jnp.int32),
        pltpu.VMEM((2 * CI, D), jnp.float32),
        pltpu.VMEM((2 * CB, D), jnp.float32),
        pltpu.SemaphoreType.DMA,
        pltpu.SemaphoreType.DMA,
        pltpu.SemaphoreType.DMA,
        pltpu.SemaphoreType.DMA,
    ],
)(_sum_body)

_sc_div = functools.partial(
    pl.kernel,
    out_type=jax.ShapeDtypeStruct((B, D), jnp.float32),
    mesh=_MESH,
    compiler_params=_PARAMS,
    scratch_types=[
        pltpu.VMEM((IPW,), jnp.int32),
        pltpu.VMEM((2 * CI2, LANES), jnp.float32),
        pltpu.VMEM((BPW, D), jnp.float32),
        pltpu.VMEM((2 * CB2, D), jnp.float32),
        pltpu.SemaphoreType.DMA,
        pltpu.SemaphoreType.DMA,
        pltpu.SemaphoreType.DMA,
        pltpu.SemaphoreType.DMA,
    ],
)(_div_body)


@jax.jit
def kernel(x, vectors):
    rs = _rowsum_table(vectors)
    xf = x.reshape(B * L)
    sums = _sc_sum(xf, vectors)
    return _sc_div(xf, rs, sums)


# speedup vs baseline: 1.0592x; 1.0592x over previous
import functools

import jax
import jax.numpy as jnp
from jax import lax
from jax.experimental import pallas as pl
from jax.experimental.pallas import tpu as pltpu
from jax.experimental.pallas import tpu_sc as plsc

VOCAB = 100000
D = 64
B = 16384
L = 50

NC = 2
NS = 16
LANES = 16
NW = NC * NS
BPW = B // NW
IPW = BPW * L
CB = 8
NCHUNK = BPW // CB
NSUPER = NCHUNK // 2
CI = CB * L
GSIZES = [128, 128, 128, 16]

RS_BLK = 4000
RS_GRID = 25


def _rowsum_table(vectors):
    def body(v_ref, o_ref):
        splat_p = (lax.broadcasted_iota(jnp.int32, (8, 128), 1) // LANES
                   == lax.broadcasted_iota(jnp.int32, (8, 128), 0)
                   ).astype(jnp.float32)
        v3 = v_ref[...].reshape(RS_BLK // 8, 8, D)
        s2 = jnp.sum(v3, axis=2)
        o_ref[...] = jax.lax.dot_general(
            s2, splat_p, (((1,), (0,)), ((), ())),
            preferred_element_type=jnp.float32).reshape(1, RS_BLK // 8, 128)

    rs = pl.pallas_call(
        body,
        grid=(RS_GRID,),
        in_specs=[pl.BlockSpec((RS_BLK, D), lambda i: (i, 0))],
        out_specs=pl.BlockSpec((1, RS_BLK // 8, 128), lambda i: (i, 0, 0)),
        out_shape=jax.ShapeDtypeStruct((RS_GRID, RS_BLK // 8, 128), jnp.float32),
    )(vectors)
    return rs.reshape(RS_GRID * RS_BLK, LANES)


def _sc_body(x_hbm, vec_hbm, rs_hbm, out_hbm,
             idx_v, rows_v, rsg_v, stage_v, sem0, sem1, osem0, osem1):
    c = lax.axis_index("c")
    s = lax.axis_index("s")
    wid = s * NC + c
    base_b = wid * BPW

    pltpu.sync_copy(x_hbm.at[pl.ds(wid * IPW, IPW)], idx_v)

    sems = [sem0, sem1]
    osems = [osem0, osem1]

    def fire(buf, ci):
        o = 0
        for g in GSIZES:
            src = idx_v.at[pl.ds(ci * CI + o, g)]
            pltpu.async_copy(vec_hbm.at[src],
                             rows_v.at[pl.ds(buf * CI + o, g)], sems[buf])
            pltpu.async_copy(rs_hbm.at[src],
                             rsg_v.at[pl.ds(buf * CI + o, g)], sems[buf])
            o += g

    def drain(buf):
        pltpu.make_async_copy(vec_hbm.at[pl.ds(0, CI)],
                              rows_v.at[pl.ds(buf * CI, CI)], sems[buf]).wait()
        pltpu.make_async_copy(rs_hbm.at[pl.ds(0, CI)],
                              rsg_v.at[pl.ds(buf * CI, CI)], sems[buf]).wait()

    one = jnp.ones((LANES,), jnp.float32)
    zero = jnp.zeros((LANES,), jnp.float32)

    def compute(buf, ci):
        def row_body(b, carry):
            r0 = buf * CI + b * L
            accs = [zero for _ in range(D // LANES)]
            cnt = zero
            for l in range(L):
                for d in range(D // LANES):
                    accs[d] = accs[d] + rows_v[r0 + l, pl.ds(d * LANES, LANES)]
                rsl = rsg_v[r0 + l, pl.ds(0, LANES)]
                cnt = cnt + jnp.where(rsl != 0.0, one, zero)
            inv = 1.0 / cnt
            for d in range(D // LANES):
                stage_v[buf * CB + b, pl.ds(d * LANES, LANES)] = accs[d] * inv
            return carry

        @pl.when(ci >= 2)
        def _():
            pltpu.make_async_copy(
                stage_v.at[pl.ds(buf * CB, CB)],
                out_hbm.at[pl.ds(0, CB)], osems[buf]).wait()

        lax.fori_loop(0, CB, row_body, 0, unroll=2)
        pltpu.async_copy(stage_v.at[pl.ds(buf * CB, CB)],
                         out_hbm.at[pl.ds(base_b + ci * CB, CB)], osems[buf])

    fire(0, 0)

    def super_body(sc, carry):
        ci0 = sc * 2
        fire(1, ci0 + 1)
        drain(0)
        compute(0, ci0)

        @pl.when(sc + 1 < NSUPER)
        def _():
            fire(0, ci0 + 2)

        drain(1)
        compute(1, ci0 + 1)
        return carry

    lax.fori_loop(0, NSUPER, super_body, 0)

    for buf in range(2):
        pltpu.make_async_copy(stage_v.at[pl.ds(buf * CB, CB)],
                              out_hbm.at[pl.ds(0, CB)], osems[buf]).wait()


_sc_call = functools.partial(
    pl.kernel,
    out_type=jax.ShapeDtypeStruct((B, D), jnp.float32),
    mesh=plsc.VectorSubcoreMesh(core_axis_name="c", subcore_axis_name="s"),
    compiler_params=pltpu.CompilerParams(use_tc_tiling_on_sc=False),
    scratch_types=[
        pltpu.VMEM((IPW,), jnp.int32),
        pltpu.VMEM((2 * CI, D), jnp.float32),
        pltpu.VMEM((2 * CI, LANES), jnp.float32),
        pltpu.VMEM((2 * CB, D), jnp.float32),
        pltpu.SemaphoreType.DMA,
        pltpu.SemaphoreType.DMA,
        pltpu.SemaphoreType.DMA,
        pltpu.SemaphoreType.DMA,
    ],
)(_sc_body)


@jax.jit
def kernel(x, vectors):
    rs = _rowsum_table(vectors)
    return _sc_call(x.reshape(B * L), vectors, rs)
